# baseline (speedup 1.0000x reference)
import functools

import jax
import jax.numpy as jnp
import numpy as np
from jax import lax
from jax.experimental import pallas as pl
from jax.experimental.pallas import tpu as pltpu

N = 50000
E = 800000
NB = 2
A = 128
ED = 64
INV2 = 1.0 / np.sqrt(2.0)
INV3 = 1.0 / np.sqrt(3.0)
INVNB = 1.0 / np.sqrt(NB)

BE = 2000
BN = 2000

_MM = functools.partial(jnp.dot, preferred_element_type=jnp.float32)


def _silu(x):
    return x * (1.0 / (1.0 + jnp.exp(-x)))


def _blockdiag(*ws):
    rows = sum(w.shape[0] for w in ws)
    cols = sum(w.shape[1] for w in ws)
    out = jnp.zeros((rows, cols), jnp.float32)
    r = c = 0
    for w in ws:
        out = lax.dynamic_update_slice(out, w, (r, c))
        r += w.shape[0]
        c += w.shape[1]
    return out


def _blocked_call(body, n_rows, blk, in_arrays, weight_arrays, out_dims):
    grid = (n_rows // blk,)

    def mk_spec(a):
        nd = a.ndim
        return pl.BlockSpec((blk,) + a.shape[1:],
                            lambda i, _nd=nd: (i,) + (0,) * (_nd - 1))

    in_specs = [mk_spec(a) for a in in_arrays]
    in_specs += [pl.BlockSpec(w.shape, lambda i, _nd=w.ndim: (0,) * _nd)
                 for w in weight_arrays]
    out_specs = [pl.BlockSpec((blk, d), lambda i: (i, 0)) for d in out_dims]
    out_shape = [jax.ShapeDtypeStruct((n_rows, d), jnp.float32) for d in out_dims]
    if len(out_dims) == 1:
        out_specs, out_shape = out_specs[0], out_shape[0]
    return pl.pallas_call(
        body,
        grid=grid,
        in_specs=in_specs,
        out_specs=out_specs,
        out_shape=out_shape,
    )(*in_arrays, *weight_arrays)


def _p1_body(m_ref, r_ref, w1_ref, w2_ref, w3_ref, out_ref):
    a = _silu(_MM(m_ref[...], w1_ref[...])) * _MM(r_ref[...], w2_ref[...])
    out_ref[...] = _silu(_MM(a, w3_ref[...]))


def _phase1(m_st, rbf, p):
    w1 = jnp.concatenate([p["W_q_m_rbf"], p["W_t_m_rbf"]], axis=1)
    w2 = jnp.concatenate([p["W_q_rbf"], p["W_t_rbf"]], axis=1)
    w3 = _blockdiag(p["W_q_m_cbf"], p["W_t_m_cbf"])
    return _blocked_call(_p1_body, E, BE, [m_st, rbf], [w1, w2, w3], [96])


def _p3_body(g_ref, c_ref, s_ref, wc_ref, wm_ref, ws_ref, wd_ref, wst_ref,
             st_ref, ts_ref):
    g = g_ref[...]
    cb = _MM(c_ref[...], wc_ref[...])
    q = jnp.concatenate([g[:, 0:32] * cb[:, 0:32], g[:, 96:128] * cb[:, 96:128]],
                        axis=1)
    sbt = _MM(s_ref[...], ws_ref[...])
    q = _silu(_MM(q, wm_ref[...])) * sbt
    xq = (q[:, 0:32] + q[:, 32:64]) * INVNB
    xt = (g[:, 32:96] * cb[:, 32:96] + g[:, 128:192] * cb[:, 128:192]) * INVNB
    xqt = _silu(_MM(jnp.concatenate([xq, xt], axis=1), wd_ref[...]))
    y = _silu(_MM(xqt, wst_ref[...]))
    st_ref[...] = y[:, 0:64] + y[:, 128:192]
    ts_ref[...] = y[:, 64:128] + y[:, 192:256]


def _phase3(g, cbf2, sbf2, p):
    wcf = jnp.concatenate([p["W_q_cbf"], p["W_t_cbf"]], axis=1)
    wc = _blockdiag(wcf, wcf)
    wm = _blockdiag(p["W_q_m_sbf"], p["W_q_m_sbf"])
    ws = _blockdiag(p["W_q_sbf"], p["W_q_sbf"])
    wd = _blockdiag(p["W_q_dir"], p["W_t_dir"])
    wst = _blockdiag(
        jnp.concatenate([p["W_q_st"], p["W_q_ts"]], axis=1),
        jnp.concatenate([p["W_t_st"], p["W_t_ts"]], axis=1))
    return _blocked_call(_p3_body, E, BE, [g, cbf2, sbf2],
                         [wc, wm, ws, wd, wst], [64, 64])


def _p5_body(m_ref, r_ref, st_ref, sw_ref, wmr_ref, wres_ref, x_ref, xa_ref):
    mr = jnp.concatenate([m_ref[...], r_ref[...]], axis=1)
    da = _MM(mr, wmr_ref[...])
    x = (_silu(da[:, 0:64]) + (st_ref[...] + sw_ref[...]) * INV2) * INV3
    wres = wres_ref[...]
    for r in range(2):
        y = _silu(_MM(x, wres[r, 0]))
        y = _silu(_MM(y, wres[r, 1]))
        x = (x + y) * INV2
    x_ref[...] = x
    xa_ref[...] = x * da[:, 64:128]


def _phase5(m_st, rbf, st_sum, sw, p):
    wmr = _blockdiag(p["W_m_dense"], p["W_a_rbf"])
    wres = jnp.stack([
        jnp.stack([p["W_res_m_0_0"], p["W_res_m_0_1"]]),
        jnp.stack([p["W_res_m_1_0"], p["W_res_m_1_1"]])])
    return _blocked_call(_p5_body, E, BE, [m_st, rbf, st_sum, sw],
                         [wmr, wres], [64, 64])


def _p7_body(hs_ref, h_ref, wa_ref, wres_ref, wself_ref,
             hout_ref, hs1_ref, hs2_ref):
    hx = _silu(_MM(hs_ref[...], wa_ref[...]))
    wres = wres_ref[...]
    for r in range(2):
        y = _silu(_MM(hx, wres[r, 0]))
        y = _silu(_MM(y, wres[r, 1]))
        hx = (hx + y) * INV2
    h_out = (h_ref[...] + hx) * INV2
    hout_ref[...] = h_out
    hs12 = _MM(h_out, wself_ref[...])
    hs1_ref[...] = hs12[:, 0:64]
    hs2_ref[...] = hs12[:, 64:128]


def _phase7(hsum, h, p):
    wres = jnp.stack([
        jnp.stack([p["W_res_a_0_0"], p["W_res_a_0_1"]]),
        jnp.stack([p["W_res_a_1_0"], p["W_res_a_1_1"]])])
    wself = jnp.concatenate([p["W_self"][0:128], p["W_self"][128:256]], axis=1)
    return _blocked_call(_p7_body, N, BN, [hsum, h],
                         [p["W_a_0"], wres, wself], [128, 64, 64])


def _p9_body(e1_ref, e2_ref, x_ref, w3_ref, wres_ref, out_ref):
    m = _silu(e1_ref[...] + e2_ref[...] + _MM(x_ref[...], w3_ref[...]))
    wres = wres_ref[...]
    for r in range(2):
        y = _silu(_MM(m, wres[r, 0]))
        y = _silu(_MM(y, wres[r, 1]))
        m = (m + y) * INV2
    out_ref[...] = m


def _phase9(e1, e2, x, p):
    wres = jnp.stack([
        jnp.stack([p["W_res_m2_0_0"], p["W_res_m2_0_1"]]),
        jnp.stack([p["W_res_m2_1_0"], p["W_res_m2_1_1"]])])
    return _blocked_call(_p9_body, E, BE, [e1, e2, x],
                         [p["W_self"][256:320], wres], [64])


def kernel(h, m_st, rbf, cbf, sbf, idx_s, idx_t, idx_swap,
           basis_idx1, basis_idx2, params):
    p = params
    idx_s = idx_s.astype(jnp.int32)
    idx_t = idx_t.astype(jnp.int32)
    idx_swap = idx_swap.astype(jnp.int32)
    b1 = basis_idx1.astype(jnp.int32).reshape(-1)
    b2 = basis_idx2.astype(jnp.int32).reshape(-1)

    AB = _phase1(m_st, rbf, p)

    g1 = AB[b1]
    g2 = AB[b2, 0:32]
    qp = (g1[:, 0:32] + g2) * INV2
    node_tab = jnp.concatenate([qp, g1[:, 32:96]], axis=1).reshape(N, 192)

    g = node_tab[idx_s]

    st_sum, ts_sum = _phase3(g, cbf.reshape(E, NB * 16),
                             sbf.reshape(E, NB * 32), p)

    sw = ts_sum[idx_swap]

    x, xa = _phase5(m_st, rbf, st_sum, sw, p)

    hsum = jax.ops.segment_sum(xa, idx_t, num_segments=N)

    h_out, hs1, hs2 = _phase7(hsum, h, p)

    e1 = hs1[idx_s]
    e2 = hs2[idx_t]

    m_new = _phase9(e1, e2, x, p)
    return h_out, m_new
